# baseline (speedup 1.0000x reference)
import functools

import jax
import jax.numpy as jnp
from jax import lax
from jax.experimental import pallas as pl
from jax.experimental.pallas import tpu as pltpu
from jax.experimental.pallas import tpu_sc as plsc

V = 39
P = 50
D = 48

NC = 2
NS = 16
NW = NC * NS

ROWS = 16384 * P
ROWS_PER_TILE = ROWS // NW
CHUNK = 512
NCHUNK = ROWS_PER_TILE // CHUNK
SUB = 128
NSUB = CHUNK // SUB
PEXT = 576
NBUF = 2


def _table_body(w_ref, pe_ref, t_ref):
    t_ref[...] = w_ref[...][:, None, :] + pe_ref[...][None, :, :]


def _build_table(W, pe2d):
    t = pl.pallas_call(
        _table_body,
        out_shape=jax.ShapeDtypeStruct((V, P, D), jnp.float32),
    )(W, pe2d)
    return t.reshape(V * P, D)


def _sc_body(x_hbm, t_hbm, out_hbm, idx_raw, idx_c, rows, p_ext,
             sem_idx0, sem_idx1, sem_gat, sem_out0, sem_out1):
    sem_idx = (sem_idx0, sem_idx1)
    sem_out = (sem_out0, sem_out1)
    wid = lax.axis_index("s") * NC + lax.axis_index("c")
    tile_base = wid * ROWS_PER_TILE

    iota = lax.iota(jnp.int32, 16)
    for s in range(PEXT // 16):
        m = (s * 16) % P
        v = iota + m
        p_ext[pl.ds(s * 16, 16)] = jnp.where(v >= P, v - P, v)

    for b in range(NBUF):
        pltpu.async_copy(
            x_hbm.at[pl.ds(tile_base + b * CHUNK, CHUNK)],
            idx_raw.at[b], sem_idx[b],
        )

    def body(c2, off):
        for b in range(NBUF):
            ch = c2 * NBUF + b
            base = tile_base + ch * CHUNK
            pltpu.make_async_copy(
                x_hbm.at[pl.ds(base, CHUNK)], idx_raw.at[b], sem_idx[b]
            ).wait()
            for s in range(CHUNK // 16):
                xv = idx_raw[b, pl.ds(s * 16, 16)]
                pv = p_ext[pl.ds(off + s * 16, 16)]
                idx_c[b, pl.ds(s * 16, 16)] = xv * P + pv
            off2 = off + (CHUNK % P)
            off = lax.select(off2 >= P, off2 - P, off2)
            @pl.when(ch + NBUF < NCHUNK)
            def _():
                pltpu.async_copy(
                    x_hbm.at[pl.ds(base + NBUF * CHUNK, CHUNK)],
                    idx_raw.at[b], sem_idx[b],
                )
            @pl.when(ch >= NBUF)
            def _():
                pltpu.make_async_copy(
                    rows.at[b],
                    out_hbm.at[pl.ds(base - NBUF * CHUNK, CHUNK)],
                    sem_out[b],
                ).wait()
            cps = [
                pltpu.async_copy(
                    t_hbm.at[idx_c.at[b, pl.ds(j * SUB, SUB)]],
                    rows.at[b, pl.ds(j * SUB, SUB)],
                    sem_gat,
                )
                for j in range(NSUB)
            ]
            for cp in cps:
                cp.wait()
            pltpu.async_copy(
                rows.at[b], out_hbm.at[pl.ds(base, CHUNK)], sem_out[b]
            )
        return off

    lax.fori_loop(0, NCHUNK // NBUF, body, jnp.int32(0))

    for b in range(NBUF):
        base = tile_base + (NCHUNK - NBUF + b) * CHUNK
        pltpu.make_async_copy(
            rows.at[b], out_hbm.at[pl.ds(base, CHUNK)], sem_out[b]
        ).wait()


@jax.jit
def _run(x_flat, table):
    mesh = plsc.VectorSubcoreMesh(core_axis_name="c", subcore_axis_name="s")
    sc = functools.partial(
        pl.kernel,
        mesh=mesh,
        out_type=jax.ShapeDtypeStruct((ROWS, D), jnp.float32),
        scratch_types=[
            pltpu.VMEM((NBUF, CHUNK), jnp.int32),
            pltpu.VMEM((NBUF, CHUNK), jnp.int32),
            pltpu.VMEM((NBUF, CHUNK, D), jnp.float32),
            pltpu.VMEM((PEXT,), jnp.int32),
            pltpu.SemaphoreType.DMA,
            pltpu.SemaphoreType.DMA,
            pltpu.SemaphoreType.DMA,
            pltpu.SemaphoreType.DMA,
            pltpu.SemaphoreType.DMA,
        ],
        compiler_params=pltpu.CompilerParams(use_tc_tiling_on_sc=False),
    )(_sc_body)
    return sc(x_flat, table)


def kernel(x, W, pe):
    x_flat = x.reshape(-1).astype(jnp.int32)
    table = _build_table(W, pe[0])
    out = _run(x_flat, table)
    return out.reshape(x.shape[0], P, D)


# speedup vs baseline: 1.0034x; 1.0034x over previous
import functools

import jax
import jax.numpy as jnp
from jax import lax
from jax.experimental import pallas as pl
from jax.experimental.pallas import tpu as pltpu
from jax.experimental.pallas import tpu_sc as plsc

V = 39
P = 50
D = 48
B = 16384

NC = 2
NS = 16
NW = NC * NS

ROWS = B * P
NSPLIT = 2
ROWS_H = ROWS // NSPLIT
CHUNK = 256
SUB = 128
NSUB = CHUNK // SUB
PEXT = 320
NBUF = 2


def _table_body(w_ref, pe_ref, t_ref):
    t_ref[...] = w_ref[...][:, None, :] + pe_ref[...][None, :, :]


def _build_table(W, pe2d):
    t = pl.pallas_call(
        _table_body,
        out_shape=jax.ShapeDtypeStruct((V, P, D), jnp.float32),
    )(W, pe2d)
    return t.reshape(V * P, D)


def _make_sc_body(rows_total):
    rows_per_tile = rows_total // NW
    nchunk = rows_per_tile // CHUNK
    assert nchunk % NBUF == 0

    def _sc_body(x_hbm, t_hbm, out_hbm, idx_raw, idx_c, rows, p_ext,
                 sem_idx0, sem_idx1, sem_gat, sem_out0, sem_out1):
        sem_idx = (sem_idx0, sem_idx1)
        sem_out = (sem_out0, sem_out1)
        wid = lax.axis_index("s") * NC + lax.axis_index("c")
        tile_base = wid * rows_per_tile

        iota = lax.iota(jnp.int32, 16)
        for s in range(PEXT // 16):
            m = (s * 16) % P
            v = iota + m
            p_ext[pl.ds(s * 16, 16)] = jnp.where(v >= P, v - P, v)

        for b in range(NBUF):
            pltpu.async_copy(
                x_hbm.at[pl.ds(tile_base + b * CHUNK, CHUNK)],
                idx_raw.at[b], sem_idx[b],
            )

        def body(c2, off):
            for b in range(NBUF):
                ch = c2 * NBUF + b
                base = tile_base + ch * CHUNK
                pltpu.make_async_copy(
                    x_hbm.at[pl.ds(base, CHUNK)], idx_raw.at[b], sem_idx[b]
                ).wait()
                for s in range(CHUNK // 16):
                    xv = idx_raw[b, pl.ds(s * 16, 16)]
                    pv = p_ext[pl.ds(off + s * 16, 16)]
                    idx_c[b, pl.ds(s * 16, 16)] = xv * P + pv
                off2 = off + (CHUNK % P)
                off = lax.select(off2 >= P, off2 - P, off2)
                @pl.when(ch + NBUF < nchunk)
                def _():
                    pltpu.async_copy(
                        x_hbm.at[pl.ds(base + NBUF * CHUNK, CHUNK)],
                        idx_raw.at[b], sem_idx[b],
                    )
                @pl.when(ch >= NBUF)
                def _():
                    pltpu.make_async_copy(
                        rows.at[b],
                        out_hbm.at[pl.ds(base - NBUF * CHUNK, CHUNK)],
                        sem_out[b],
                    ).wait()
                cps = [
                    pltpu.async_copy(
                        t_hbm.at[idx_c.at[b, pl.ds(j * SUB, SUB)]],
                        rows.at[b, pl.ds(j * SUB, SUB)],
                        sem_gat,
                    )
                    for j in range(NSUB)
                ]
                for cp in cps:
                    cp.wait()
                pltpu.async_copy(
                    rows.at[b], out_hbm.at[pl.ds(base, CHUNK)], sem_out[b]
                )
            return off

        lax.fori_loop(0, nchunk // NBUF, body, jnp.int32(0))

        for b in range(NBUF):
            base = tile_base + (nchunk - NBUF + b) * CHUNK
            pltpu.make_async_copy(
                rows.at[b], out_hbm.at[pl.ds(base, CHUNK)], sem_out[b]
            ).wait()

    return _sc_body


@jax.jit
def _run(x_flat, table):
    mesh = plsc.VectorSubcoreMesh(core_axis_name="c", subcore_axis_name="s")
    sc = functools.partial(
        pl.kernel,
        mesh=mesh,
        out_type=jax.ShapeDtypeStruct((ROWS_H, D), jnp.float32),
        scratch_types=[
            pltpu.VMEM((NBUF, CHUNK), jnp.int32),
            pltpu.VMEM((NBUF, CHUNK), jnp.int32),
            pltpu.VMEM((NBUF, CHUNK, D), jnp.float32),
            pltpu.VMEM((PEXT,), jnp.int32),
            pltpu.SemaphoreType.DMA,
            pltpu.SemaphoreType.DMA,
            pltpu.SemaphoreType.DMA,
            pltpu.SemaphoreType.DMA,
            pltpu.SemaphoreType.DMA,
        ],
        compiler_params=pltpu.CompilerParams(use_tc_tiling_on_sc=False),
    )(_make_sc_body(ROWS_H))
    halves = [
        sc(x_flat[i * ROWS_H:(i + 1) * ROWS_H], table)
        for i in range(NSPLIT)
    ]
    return jnp.concatenate(
        [h.reshape(B // NSPLIT, P, D) for h in halves], axis=0
    )


def kernel(x, W, pe):
    x_flat = x.reshape(-1).astype(jnp.int32)
    table = _build_table(W, pe[0])
    return _run(x_flat, table)


# speedup vs baseline: 1.0903x; 1.0867x over previous
import functools

import jax
import jax.numpy as jnp
from jax import lax
from jax.experimental import pallas as pl
from jax.experimental.pallas import tpu as pltpu
from jax.experimental.pallas import tpu_sc as plsc

V = 39
P = 50
D = 48
B = 16384

NC = 2
NS = 16
NW = NC * NS

ROWS = B * P
NSPLIT = 4
ROWS_H = ROWS // NSPLIT
CHUNK = 128
SUB = 128
NSUB = CHUNK // SUB
PEXT = 192
NBUF = 2


def _table_body(w_ref, pe_ref, t_ref):
    t_ref[...] = w_ref[...][:, None, :] + pe_ref[...][None, :, :]


def _build_table(W, pe2d):
    t = pl.pallas_call(
        _table_body,
        out_shape=jax.ShapeDtypeStruct((V, P, D), jnp.float32),
    )(W, pe2d)
    return t.reshape(V * P, D)


def _make_sc_body(rows_total):
    rows_per_tile = rows_total // NW
    nchunk = rows_per_tile // CHUNK
    assert nchunk % NBUF == 0

    def _sc_body(x_hbm, t_hbm, out_hbm, idx_raw, idx_c, rows, p_ext,
                 sem_idx0, sem_idx1, sem_gat, sem_out0, sem_out1):
        sem_idx = (sem_idx0, sem_idx1)
        sem_out = (sem_out0, sem_out1)
        wid = lax.axis_index("s") * NC + lax.axis_index("c")
        tile_base = wid * rows_per_tile

        iota = lax.iota(jnp.int32, 16)
        for s in range(PEXT // 16):
            m = (s * 16) % P
            v = iota + m
            p_ext[pl.ds(s * 16, 16)] = jnp.where(v >= P, v - P, v)

        for b in range(NBUF):
            pltpu.async_copy(
                x_hbm.at[pl.ds(tile_base + b * CHUNK, CHUNK)],
                idx_raw.at[b], sem_idx[b],
            )

        def body(c2, off):
            for b in range(NBUF):
                ch = c2 * NBUF + b
                base = tile_base + ch * CHUNK
                pltpu.make_async_copy(
                    x_hbm.at[pl.ds(base, CHUNK)], idx_raw.at[b], sem_idx[b]
                ).wait()
                for s in range(CHUNK // 16):
                    xv = idx_raw[b, pl.ds(s * 16, 16)]
                    pv = p_ext[pl.ds(off + s * 16, 16)]
                    idx_c[b, pl.ds(s * 16, 16)] = xv * P + pv
                off2 = off + (CHUNK % P)
                off = lax.select(off2 >= P, off2 - P, off2)
                @pl.when(ch + NBUF < nchunk)
                def _():
                    pltpu.async_copy(
                        x_hbm.at[pl.ds(base + NBUF * CHUNK, CHUNK)],
                        idx_raw.at[b], sem_idx[b],
                    )
                @pl.when(ch >= NBUF)
                def _():
                    pltpu.make_async_copy(
                        rows.at[b],
                        out_hbm.at[pl.ds(base - NBUF * CHUNK, CHUNK)],
                        sem_out[b],
                    ).wait()
                cps = [
                    pltpu.async_copy(
                        t_hbm.at[idx_c.at[b, pl.ds(j * SUB, SUB)]],
                        rows.at[b, pl.ds(j * SUB, SUB)],
                        sem_gat,
                    )
                    for j in range(NSUB)
                ]
                for cp in cps:
                    cp.wait()
                pltpu.async_copy(
                    rows.at[b], out_hbm.at[pl.ds(base, CHUNK)], sem_out[b]
                )
            return off

        lax.fori_loop(0, nchunk // NBUF, body, jnp.int32(0))

        for b in range(NBUF):
            base = tile_base + (nchunk - NBUF + b) * CHUNK
            pltpu.make_async_copy(
                rows.at[b], out_hbm.at[pl.ds(base, CHUNK)], sem_out[b]
            ).wait()

    return _sc_body


@jax.jit
def _run(x_flat, table):
    mesh = plsc.VectorSubcoreMesh(core_axis_name="c", subcore_axis_name="s")
    sc = functools.partial(
        pl.kernel,
        mesh=mesh,
        out_type=jax.ShapeDtypeStruct((ROWS_H, D), jnp.float32),
        scratch_types=[
            pltpu.VMEM((NBUF, CHUNK), jnp.int32),
            pltpu.VMEM((NBUF, CHUNK), jnp.int32),
            pltpu.VMEM((NBUF, CHUNK, D), jnp.float32),
            pltpu.VMEM((PEXT,), jnp.int32),
            pltpu.SemaphoreType.DMA,
            pltpu.SemaphoreType.DMA,
            pltpu.SemaphoreType.DMA,
            pltpu.SemaphoreType.DMA,
            pltpu.SemaphoreType.DMA,
        ],
        compiler_params=pltpu.CompilerParams(use_tc_tiling_on_sc=False),
    )(_make_sc_body(ROWS_H))
    halves = [
        sc(x_flat[i * ROWS_H:(i + 1) * ROWS_H], table)
        for i in range(NSPLIT)
    ]
    return jnp.concatenate(
        [h.reshape(B // NSPLIT, P, D) for h in halves], axis=0
    )


def kernel(x, W, pe):
    x_flat = x.reshape(-1).astype(jnp.int32)
    table = _build_table(W, pe[0])
    return _run(x_flat, table)


# speedup vs baseline: 1.2398x; 1.1371x over previous
import functools

import jax
import jax.numpy as jnp
from jax import lax
from jax.experimental import pallas as pl
from jax.experimental.pallas import tpu as pltpu
from jax.experimental.pallas import tpu_sc as plsc

V = 39
P = 50
D = 48
B = 16384

NC = 2
NS = 16
NW = NC * NS

ROWS = B * P
NSPLIT = 8
ROWS_H = ROWS // NSPLIT
CHUNK = 64
SUB = 128
NSUB = CHUNK // SUB
PEXT = 192
NBUF = 2


def _table_body(w_ref, pe_ref, t_ref):
    t_ref[...] = w_ref[...][:, None, :] + pe_ref[...][None, :, :]


def _build_table(W, pe2d):
    t = pl.pallas_call(
        _table_body,
        out_shape=jax.ShapeDtypeStruct((V, P, D), jnp.float32),
    )(W, pe2d)
    return t.reshape(V * P, D)


def _make_sc_body(rows_total):
    rows_per_tile = rows_total // NW
    nchunk = rows_per_tile // CHUNK
    assert nchunk % NBUF == 0

    def _sc_body(x_hbm, t_hbm, out_hbm, idx_raw, idx_c, rows, p_ext,
                 sem_idx0, sem_idx1, sem_gat, sem_out0, sem_out1):
        sem_idx = (sem_idx0, sem_idx1)
        sem_out = (sem_out0, sem_out1)
        wid = lax.axis_index("s") * NC + lax.axis_index("c")
        tile_base = wid * rows_per_tile

        iota = lax.iota(jnp.int32, 16)
        for s in range(PEXT // 16):
            m = (s * 16) % P
            v = iota + m
            p_ext[pl.ds(s * 16, 16)] = jnp.where(v >= P, v - P, v)

        for b in range(NBUF):
            pltpu.async_copy(
                x_hbm.at[pl.ds(tile_base + b * CHUNK, CHUNK)],
                idx_raw.at[b], sem_idx[b],
            )

        def body(c2, off):
            for b in range(NBUF):
                ch = c2 * NBUF + b
                base = tile_base + ch * CHUNK
                pltpu.make_async_copy(
                    x_hbm.at[pl.ds(base, CHUNK)], idx_raw.at[b], sem_idx[b]
                ).wait()
                for s in range(CHUNK // 16):
                    xv = idx_raw[b, pl.ds(s * 16, 16)]
                    pv = p_ext[pl.ds(off + s * 16, 16)]
                    idx_c[b, pl.ds(s * 16, 16)] = xv * P + pv
                off2 = off + (CHUNK % P)
                off = lax.select(off2 >= P, off2 - P, off2)
                @pl.when(ch + NBUF < nchunk)
                def _():
                    pltpu.async_copy(
                        x_hbm.at[pl.ds(base + NBUF * CHUNK, CHUNK)],
                        idx_raw.at[b], sem_idx[b],
                    )
                @pl.when(ch >= NBUF)
                def _():
                    pltpu.make_async_copy(
                        rows.at[b],
                        out_hbm.at[pl.ds(base - NBUF * CHUNK, CHUNK)],
                        sem_out[b],
                    ).wait()
                cps = [
                    pltpu.async_copy(
                        t_hbm.at[idx_c.at[b, pl.ds(j * SUB, SUB)]],
                        rows.at[b, pl.ds(j * SUB, SUB)],
                        sem_gat,
                    )
                    for j in range(NSUB)
                ]
                for cp in cps:
                    cp.wait()
                pltpu.async_copy(
                    rows.at[b], out_hbm.at[pl.ds(base, CHUNK)], sem_out[b]
                )
            return off

        lax.fori_loop(0, nchunk // NBUF, body, jnp.int32(0))

        for b in range(NBUF):
            base = tile_base + (nchunk - NBUF + b) * CHUNK
            pltpu.make_async_copy(
                rows.at[b], out_hbm.at[pl.ds(base, CHUNK)], sem_out[b]
            ).wait()

    return _sc_body


@jax.jit
def _run(x_flat, table):
    mesh = plsc.VectorSubcoreMesh(core_axis_name="c", subcore_axis_name="s")
    sc = functools.partial(
        pl.kernel,
        mesh=mesh,
        out_type=jax.ShapeDtypeStruct((ROWS_H, D), jnp.float32),
        scratch_types=[
            pltpu.VMEM((NBUF, CHUNK), jnp.int32),
            pltpu.VMEM((NBUF, CHUNK), jnp.int32),
            pltpu.VMEM((NBUF, CHUNK, D), jnp.float32),
            pltpu.VMEM((PEXT,), jnp.int32),
            pltpu.SemaphoreType.DMA,
            pltpu.SemaphoreType.DMA,
            pltpu.SemaphoreType.DMA,
            pltpu.SemaphoreType.DMA,
            pltpu.SemaphoreType.DMA,
        ],
        compiler_params=pltpu.CompilerParams(use_tc_tiling_on_sc=False),
    )(_make_sc_body(ROWS_H))
    halves = [
        sc(x_flat[i * ROWS_H:(i + 1) * ROWS_H], table)
        for i in range(NSPLIT)
    ]
    return jnp.concatenate(
        [h.reshape(B // NSPLIT, P, D) for h in halves], axis=0
    )


def kernel(x, W, pe):
    x_flat = x.reshape(-1).astype(jnp.int32)
    table = _build_table(W, pe[0])
    return _run(x_flat, table)
